# baseline (speedup 1.0000x reference)
import jax
import jax.numpy as jnp
from jax.experimental import pallas as pl
from jax.experimental.pallas import tpu as pltpu

N = 4096
D = 128
H = 64
G = 128
C = 10

BM = 512
BK = 512
M = N // BM
K = N // BK


def _hamilton(kernel):
    r, i, j, k = jnp.split(kernel, 4, axis=1)
    r2 = jnp.concatenate([r, -i, -j, -k], axis=0)
    i2 = jnp.concatenate([i, r, -k, j], axis=0)
    j2 = jnp.concatenate([j, k, r, -i], axis=0)
    k2 = jnp.concatenate([k, -j, i, r], axis=0)
    return jnp.concatenate([r2, i2, j2, k2], axis=1)


def _qgnn_kernel(adj_ref, x0_ref, gp_ref, h0_ref, h1_ref, pw0_ref, pw1_ref,
                 pb_ref, scores_ref, adj_sc, sup_sc, x_sc, acc_sc):
    l = pl.program_id(0)
    m = pl.program_id(1)
    k = pl.program_id(2)

    @pl.when((l == 0) & (m == 0) & (k == 0))
    def _():
        scores_ref[...] = jnp.broadcast_to(pb_ref[...], scores_ref.shape)

    @pl.when((l == 0) & (m == 0))
    def _():
        sup_sc[pl.ds(k * BK, BK), :] = jnp.dot(
            x0_ref[...], h0_ref[...],
            preferred_element_type=jnp.float32).astype(jnp.bfloat16)

    @pl.when((l == 1) & (m == 0))
    def _():
        sup_sc[pl.ds(k * BK, BK), :] = jnp.dot(
            x_sc[pl.ds(k * BK, BK), :], h1_ref[...],
            preferred_element_type=jnp.float32).astype(jnp.bfloat16)

    @pl.when(k == 0)
    def _():
        acc_sc[...] = jnp.zeros_like(acc_sc)

    sup_blk = sup_sc[pl.ds(k * BK, BK), :]

    @pl.when(l == 0)
    def _():
        a_bf = adj_ref[...].astype(jnp.bfloat16)
        adj_sc[m * K + k] = a_bf
        acc_sc[...] += jnp.dot(a_bf, sup_blk,
                               preferred_element_type=jnp.float32)

    @pl.when(l == 1)
    def _():
        acc_sc[...] += jnp.dot(adj_sc[m * K + k], sup_blk,
                               preferred_element_type=jnp.float32)

    @pl.when(k == K - 1)
    def _():
        xt = jnp.tanh(acc_sc[...])

        @pl.when(l == 0)
        def _():
            x_sc[pl.ds(m * BM, BM), :] = xt

        ge = jnp.dot(gp_ref[...], xt, preferred_element_type=jnp.float32)
        pw = jnp.where(l == 0, pw0_ref[...], pw1_ref[...])
        scores_ref[...] += jnp.dot(ge, pw, preferred_element_type=jnp.float32)


def kernel(Adj_block, X_concat, graph_pool, W0, W1, P0_w, P0_b, P1_w, P1_b):
    h0 = _hamilton(W0)
    h1 = _hamilton(W1)
    pb = (P0_b + P1_b).reshape(1, C)

    grid = (2, M, K)
    scores = pl.pallas_call(
        _qgnn_kernel,
        grid=grid,
        in_specs=[
            pl.BlockSpec((BM, BK),
                         lambda l, m, k: (jnp.where(l == 0, m, 0),
                                          jnp.where(l == 0, k, 0))),
            pl.BlockSpec((BK, D),
                         lambda l, m, k: (jnp.where((l == 0) & (m == 0), k, 0),
                                          0)),
            pl.BlockSpec((G, BM), lambda l, m, k: (0, m)),
            pl.BlockSpec((D, H), lambda l, m, k: (0, 0)),
            pl.BlockSpec((H, H), lambda l, m, k: (0, 0)),
            pl.BlockSpec((H, C), lambda l, m, k: (0, 0)),
            pl.BlockSpec((H, C), lambda l, m, k: (0, 0)),
            pl.BlockSpec((1, C), lambda l, m, k: (0, 0)),
        ],
        out_specs=pl.BlockSpec((G, C), lambda l, m, k: (0, 0)),
        out_shape=jax.ShapeDtypeStruct((G, C), jnp.float32),
        scratch_shapes=[
            pltpu.VMEM((M * K, BM, BK), jnp.bfloat16),
            pltpu.VMEM((N, H), jnp.bfloat16),
            pltpu.VMEM((N, H), jnp.float32),
            pltpu.VMEM((BM, H), jnp.float32),
        ],
        compiler_params=pltpu.CompilerParams(
            dimension_semantics=("arbitrary", "arbitrary", "arbitrary"),
        ),
    )(Adj_block, X_concat, graph_pool, h0, h1, P0_w, P1_w, pb)
    return scores


# speedup vs baseline: 1.7429x; 1.7429x over previous
import jax
import jax.numpy as jnp
from jax.experimental import pallas as pl
from jax.experimental.pallas import tpu as pltpu

N = 4096
D = 128
H = 64
G = 128
C = 10

BM = 256
M = N // BM


def _hamilton(kernel):
    r, i, j, k = jnp.split(kernel, 4, axis=1)
    r2 = jnp.concatenate([r, -i, -j, -k], axis=0)
    i2 = jnp.concatenate([i, r, -k, j], axis=0)
    j2 = jnp.concatenate([j, k, r, -i], axis=0)
    k2 = jnp.concatenate([k, -j, i, r], axis=0)
    return jnp.concatenate([r2, i2, j2, k2], axis=1)


def _qgnn_kernel(adj_ref, x0_ref, gp_ref, h0_ref, h1_ref, pw0_ref, pw1_ref,
                 pb_ref, scores_ref, adj_sc, sup_sc, x_sc, xt_sc):
    l = pl.program_id(0)
    m = pl.program_id(1)

    @pl.when((l == 0) & (m == 0))
    def _():
        scores_ref[...] = jnp.broadcast_to(pb_ref[...], scores_ref.shape)

    @pl.when((l == 0) & (m == 0))
    def _():
        sup_sc[...] = jnp.dot(
            x0_ref[...], h0_ref[...],
            preferred_element_type=jnp.float32).astype(jnp.bfloat16)

    @pl.when((l == 1) & (m == 0))
    def _():
        sup_sc[...] = jnp.dot(
            x_sc[...], h1_ref[...],
            preferred_element_type=jnp.float32).astype(jnp.bfloat16)

    @pl.when(l == 0)
    def _():
        adj_sc[m] = adj_ref[...].astype(jnp.bfloat16)
        xt = jnp.tanh(jnp.dot(adj_sc[m], sup_sc[...],
                              preferred_element_type=jnp.float32))
        x_sc[pl.ds(m * BM, BM), :] = xt
        xt_sc[...] = xt

    @pl.when(l == 1)
    def _():
        xt_sc[...] = jnp.tanh(jnp.dot(adj_sc[m], sup_sc[...],
                                      preferred_element_type=jnp.float32))

    ge = jnp.dot(gp_ref[...], xt_sc[...], preferred_element_type=jnp.float32)
    pw = jnp.where(l == 0, pw0_ref[...], pw1_ref[...])
    scores_ref[...] += jnp.dot(ge, pw, preferred_element_type=jnp.float32)


def kernel(Adj_block, X_concat, graph_pool, W0, W1, P0_w, P0_b, P1_w, P1_b):
    h0 = _hamilton(W0)
    h1 = _hamilton(W1)
    pb = (P0_b + P1_b).reshape(1, C)

    scores = pl.pallas_call(
        _qgnn_kernel,
        grid=(2, M),
        in_specs=[
            pl.BlockSpec((BM, N), lambda l, m: (jnp.where(l == 0, m, 0), 0)),
            pl.BlockSpec((N, D), lambda l, m: (0, 0)),
            pl.BlockSpec((G, BM), lambda l, m: (0, m)),
            pl.BlockSpec((D, H), lambda l, m: (0, 0)),
            pl.BlockSpec((H, H), lambda l, m: (0, 0)),
            pl.BlockSpec((H, C), lambda l, m: (0, 0)),
            pl.BlockSpec((H, C), lambda l, m: (0, 0)),
            pl.BlockSpec((1, C), lambda l, m: (0, 0)),
        ],
        out_specs=pl.BlockSpec((G, C), lambda l, m: (0, 0)),
        out_shape=jax.ShapeDtypeStruct((G, C), jnp.float32),
        scratch_shapes=[
            pltpu.VMEM((M, BM, N), jnp.bfloat16),
            pltpu.VMEM((N, H), jnp.bfloat16),
            pltpu.VMEM((N, H), jnp.float32),
            pltpu.VMEM((BM, H), jnp.float32),
        ],
        compiler_params=pltpu.CompilerParams(
            dimension_semantics=("arbitrary", "arbitrary"),
        ),
    )(Adj_block, X_concat, graph_pool, h0, h1, P0_w, P1_w, pb)
    return scores


# speedup vs baseline: 1.9613x; 1.1253x over previous
import jax
import jax.numpy as jnp
from jax.experimental import pallas as pl
from jax.experimental.pallas import tpu as pltpu

N = 4096
D = 128
H = 64
G = 128
C = 10

BM = 512
M = N // BM


def _hamilton(kernel):
    r, i, j, k = jnp.split(kernel, 4, axis=1)
    r2 = jnp.concatenate([r, -i, -j, -k], axis=0)
    i2 = jnp.concatenate([i, r, -k, j], axis=0)
    j2 = jnp.concatenate([j, k, r, -i], axis=0)
    k2 = jnp.concatenate([k, -j, i, r], axis=0)
    return jnp.concatenate([r2, i2, j2, k2], axis=1)


def _qgnn_kernel(adj_ref, x0_ref, gp_ref, h0_ref, h1_ref, pw0_ref, pw1_ref,
                 pb_ref, scores_ref, adj_sc, sup_sc, x_sc, xt_sc):
    l = pl.program_id(0)
    m = pl.program_id(1)

    @pl.when((l == 0) & (m == 0))
    def _():
        scores_ref[...] = jnp.broadcast_to(pb_ref[...], scores_ref.shape)

    @pl.when((l == 0) & (m == 0))
    def _():
        sup_sc[...] = jnp.dot(
            x0_ref[...], h0_ref[...],
            preferred_element_type=jnp.float32).astype(jnp.bfloat16)

    @pl.when((l == 1) & (m == 0))
    def _():
        sup_sc[...] = jnp.dot(
            x_sc[...], h1_ref[...],
            preferred_element_type=jnp.float32).astype(jnp.bfloat16)

    @pl.when(l == 0)
    def _():
        a_bf = adj_ref[...].astype(jnp.bfloat16)
        adj_sc[m] = a_bf
        xt = jnp.tanh(jnp.dot(a_bf, sup_sc[...],
                              preferred_element_type=jnp.float32))
        x_sc[pl.ds(m * BM, BM), :] = xt
        xt_sc[...] = xt

    @pl.when(l == 1)
    def _():
        xt_sc[...] = jnp.tanh(jnp.dot(adj_sc[m], sup_sc[...],
                                      preferred_element_type=jnp.float32))

    ge = jnp.dot(gp_ref[...], xt_sc[...], preferred_element_type=jnp.float32)
    pw = jnp.where(l == 0, pw0_ref[...], pw1_ref[...])
    scores_ref[...] += jnp.dot(ge, pw, preferred_element_type=jnp.float32)


def kernel(Adj_block, X_concat, graph_pool, W0, W1, P0_w, P0_b, P1_w, P1_b):
    h0 = _hamilton(W0)
    h1 = _hamilton(W1)
    pb = (P0_b + P1_b).reshape(1, C)

    scores = pl.pallas_call(
        _qgnn_kernel,
        grid=(2, M),
        in_specs=[
            pl.BlockSpec((BM, N), lambda l, m: (jnp.where(l == 0, m, 0), 0)),
            pl.BlockSpec((N, D), lambda l, m: (0, 0)),
            pl.BlockSpec((G, BM), lambda l, m: (0, m)),
            pl.BlockSpec((D, H), lambda l, m: (0, 0)),
            pl.BlockSpec((H, H), lambda l, m: (0, 0)),
            pl.BlockSpec((H, C), lambda l, m: (0, 0)),
            pl.BlockSpec((H, C), lambda l, m: (0, 0)),
            pl.BlockSpec((1, C), lambda l, m: (0, 0)),
        ],
        out_specs=pl.BlockSpec((G, C), lambda l, m: (0, 0)),
        out_shape=jax.ShapeDtypeStruct((G, C), jnp.float32),
        scratch_shapes=[
            pltpu.VMEM((M, BM, N), jnp.bfloat16),
            pltpu.VMEM((N, H), jnp.bfloat16),
            pltpu.VMEM((N, H), jnp.float32),
            pltpu.VMEM((BM, H), jnp.float32),
        ],
        compiler_params=pltpu.CompilerParams(
            dimension_semantics=("arbitrary", "arbitrary"),
        ),
    )(Adj_block, X_concat, graph_pool, h0, h1, P0_w, P1_w, pb)
    return scores


# speedup vs baseline: 1.9656x; 1.0022x over previous
import jax
import jax.numpy as jnp
from jax.experimental import pallas as pl
from jax.experimental.pallas import tpu as pltpu

N = 4096
D = 128
H = 64
G = 128
C = 10

BM = 512
M = N // BM


def _hamilton(kernel):
    r, i, j, k = jnp.split(kernel, 4, axis=1)
    r2 = jnp.concatenate([r, -i, -j, -k], axis=0)
    i2 = jnp.concatenate([i, r, -k, j], axis=0)
    j2 = jnp.concatenate([j, k, r, -i], axis=0)
    k2 = jnp.concatenate([k, -j, i, r], axis=0)
    return jnp.concatenate([r2, i2, j2, k2], axis=1)


def _qgnn_kernel(adj_ref, x0_ref, gp_ref, h0_ref, h1_ref, pw0_ref, pw1_ref,
                 pb_ref, scores_ref, adj_sc, sup_sc, x_sc):
    l = pl.program_id(0)
    m = pl.program_id(1)

    def pool_head(xt, pw_ref):
        gp_bf = gp_ref[...].astype(jnp.bfloat16)
        ge = jnp.dot(gp_bf, xt.astype(jnp.bfloat16),
                     preferred_element_type=jnp.float32)
        scores_ref[...] += jnp.dot(ge, pw_ref[...],
                                   preferred_element_type=jnp.float32)

    @pl.when((l == 0) & (m == 0))
    def _():
        scores_ref[...] = jnp.broadcast_to(pb_ref[...], scores_ref.shape)

    @pl.when((l == 0) & (m == 0))
    def _():
        sup_sc[...] = jnp.dot(
            x0_ref[...], h0_ref[...],
            preferred_element_type=jnp.float32).astype(jnp.bfloat16)

    @pl.when((l == 1) & (m == 0))
    def _():
        sup_sc[...] = jnp.dot(
            x_sc[...], h1_ref[...],
            preferred_element_type=jnp.float32).astype(jnp.bfloat16)

    @pl.when(l == 0)
    def _():
        a_bf = adj_ref[...].astype(jnp.bfloat16)
        adj_sc[m] = a_bf
        xt = jnp.tanh(jnp.dot(a_bf, sup_sc[...],
                              preferred_element_type=jnp.float32))
        x_sc[pl.ds(m * BM, BM), :] = xt
        pool_head(xt, pw0_ref)

    @pl.when(l == 1)
    def _():
        xt = jnp.tanh(jnp.dot(adj_sc[m], sup_sc[...],
                              preferred_element_type=jnp.float32))
        pool_head(xt, pw1_ref)


def kernel(Adj_block, X_concat, graph_pool, W0, W1, P0_w, P0_b, P1_w, P1_b):
    h0 = _hamilton(W0)
    h1 = _hamilton(W1)
    pb = (P0_b + P1_b).reshape(1, C)

    scores = pl.pallas_call(
        _qgnn_kernel,
        grid=(2, M),
        in_specs=[
            pl.BlockSpec((BM, N), lambda l, m: (jnp.where(l == 0, m, 0), 0)),
            pl.BlockSpec((N, D), lambda l, m: (0, 0)),
            pl.BlockSpec((G, BM), lambda l, m: (0, m)),
            pl.BlockSpec((D, H), lambda l, m: (0, 0)),
            pl.BlockSpec((H, H), lambda l, m: (0, 0)),
            pl.BlockSpec((H, C), lambda l, m: (0, 0)),
            pl.BlockSpec((H, C), lambda l, m: (0, 0)),
            pl.BlockSpec((1, C), lambda l, m: (0, 0)),
        ],
        out_specs=pl.BlockSpec((G, C), lambda l, m: (0, 0)),
        out_shape=jax.ShapeDtypeStruct((G, C), jnp.float32),
        scratch_shapes=[
            pltpu.VMEM((M, BM, N), jnp.bfloat16),
            pltpu.VMEM((N, H), jnp.bfloat16),
            pltpu.VMEM((N, H), jnp.float32),
        ],
        compiler_params=pltpu.CompilerParams(
            dimension_semantics=("arbitrary", "arbitrary"),
        ),
    )(Adj_block, X_concat, graph_pool, h0, h1, P0_w, P1_w, pb)
    return scores


# speedup vs baseline: 1.9671x; 1.0008x over previous
import jax
import jax.numpy as jnp
from jax.experimental import pallas as pl
from jax.experimental.pallas import tpu as pltpu

N = 4096
D = 128
H = 64
G = 128
C = 10

BM = 512
M = N // BM


def _hamilton(kernel):
    r, i, j, k = jnp.split(kernel, 4, axis=1)
    r2 = jnp.concatenate([r, -i, -j, -k], axis=0)
    i2 = jnp.concatenate([i, r, -k, j], axis=0)
    j2 = jnp.concatenate([j, k, r, -i], axis=0)
    k2 = jnp.concatenate([k, -j, i, r], axis=0)
    return jnp.concatenate([r2, i2, j2, k2], axis=1)


def _qgnn_kernel(adj_ref, x0_ref, gp_ref, h0_ref, h1_ref, pw0_ref, pw1_ref,
                 pb_ref, scores_ref, adj_sc, sup_sc, x_sc):
    l = pl.program_id(0)
    m = pl.program_id(1)

    def pool_head(xt, pw_ref):
        gp_bf = gp_ref[...].astype(jnp.bfloat16)
        ge = jnp.dot(gp_bf, xt.astype(jnp.bfloat16),
                     preferred_element_type=jnp.float32)
        scores_ref[...] += jnp.dot(ge, pw_ref[...],
                                   preferred_element_type=jnp.float32)

    @pl.when((l == 0) & (m == 0))
    def _():
        scores_ref[...] = jnp.broadcast_to(pb_ref[...], scores_ref.shape)

    @pl.when((l == 0) & (m == 0))
    def _():
        sup_sc[...] = jnp.dot(
            x0_ref[...], h0_ref[...],
            preferred_element_type=jnp.float32).astype(jnp.bfloat16)

    @pl.when((l == 1) & (m == 0))
    def _():
        sup_sc[...] = jnp.dot(
            x_sc[...], h1_ref[...],
            preferred_element_type=jnp.float32).astype(jnp.bfloat16)

    @pl.when(l == 0)
    def _():
        CH = 4
        W = N // CH
        acc = jnp.zeros((BM, H), dtype=jnp.float32)
        for j in range(CH):
            a_bf = adj_ref[:, j * W:(j + 1) * W].astype(jnp.bfloat16)
            adj_sc[m, :, j * W:(j + 1) * W] = a_bf
            acc += jnp.dot(a_bf, sup_sc[j * W:(j + 1) * W, :],
                           preferred_element_type=jnp.float32)
        xt = jnp.tanh(acc)
        x_sc[pl.ds(m * BM, BM), :] = xt
        pool_head(xt, pw0_ref)

    @pl.when(l == 1)
    def _():
        xt = jnp.tanh(jnp.dot(adj_sc[m], sup_sc[...],
                              preferred_element_type=jnp.float32))
        pool_head(xt, pw1_ref)


def kernel(Adj_block, X_concat, graph_pool, W0, W1, P0_w, P0_b, P1_w, P1_b):
    h0 = _hamilton(W0)
    h1 = _hamilton(W1)
    pb = (P0_b + P1_b).reshape(1, C)

    scores = pl.pallas_call(
        _qgnn_kernel,
        grid=(2, M),
        in_specs=[
            pl.BlockSpec((BM, N), lambda l, m: (jnp.where(l == 0, m, 0), 0)),
            pl.BlockSpec((N, D), lambda l, m: (0, 0)),
            pl.BlockSpec((G, BM), lambda l, m: (0, m)),
            pl.BlockSpec((D, H), lambda l, m: (0, 0)),
            pl.BlockSpec((H, H), lambda l, m: (0, 0)),
            pl.BlockSpec((H, C), lambda l, m: (0, 0)),
            pl.BlockSpec((H, C), lambda l, m: (0, 0)),
            pl.BlockSpec((1, C), lambda l, m: (0, 0)),
        ],
        out_specs=pl.BlockSpec((G, C), lambda l, m: (0, 0)),
        out_shape=jax.ShapeDtypeStruct((G, C), jnp.float32),
        scratch_shapes=[
            pltpu.VMEM((M, BM, N), jnp.bfloat16),
            pltpu.VMEM((N, H), jnp.bfloat16),
            pltpu.VMEM((N, H), jnp.float32),
        ],
        compiler_params=pltpu.CompilerParams(
            dimension_semantics=("arbitrary", "arbitrary"),
        ),
    )(Adj_block, X_concat, graph_pool, h0, h1, P0_w, P1_w, pb)
    return scores
